# baseline (speedup 1.0000x reference)
import functools

import jax
import jax.numpy as jnp
from jax import lax
from jax.experimental import pallas as pl
from jax.experimental.pallas import tpu as pltpu
from jax.experimental.pallas import tpu_sc as plsc

_C = 128
_NW = 32
_L = 16
_HI = -65536

_DNUMS = lax.GatherDimensionNumbers(
    offset_dims=(), collapsed_slice_dims=(0,), start_index_map=(0,))


def _lane_shuffle(v, perm):
    return lax.gather(v, perm[:, None], _DNUMS, slice_sizes=(1,),
                      mode=lax.GatherScatterMode.PROMISE_IN_BOUNDS)


def _f32(x):
    return lax.bitcast_convert_type(x, jnp.float32)


def _dot_chunk(urows, vrows, obuf, DW):
    lane = lax.iota(jnp.int32, _L)

    def g_body(g, carry):
        row0 = g * _L

        def e_body(e, tot):
            row = row0 + 2 * e
            for k in range(2):
                r = row + k
                uw = urows[r, pl.ds(0, _L)]
                vw = vrows[r, pl.ds(0, _L)]
                acc0 = _f32(uw << 16) * _f32(vw << 16)
                acc1 = _f32(uw & _HI) * _f32(vw & _HI)
                for j in range(1, DW // _L):
                    uw = urows[r, pl.ds(j * _L, _L)]
                    vw = vrows[r, pl.ds(j * _L, _L)]
                    acc0 = acc0 + _f32(uw << 16) * _f32(vw << 16)
                    acc1 = acc1 + _f32(uw & _HI) * _f32(vw & _HI)
                acc = acc0 + acc1
                for s in (1, 2, 4, 8):
                    acc = acc + _lane_shuffle(acc, lane ^ s)
                tot = jnp.where(lane == 2 * e + k, acc, tot)
            return tot

        tot = lax.fori_loop(0, _L // 2, e_body, jnp.zeros((_L,), jnp.float32))
        obuf[pl.ds(row0, _L)] = tot
        return carry

    lax.fori_loop(0, _C // _L, g_body, 0)


def kernel(new_ft, raw_ft, edge_index):
    N, D = new_ft.shape
    E = edge_index.shape[1]
    DW = D // 2
    assert E % _C == 0
    num_chunks = E // _C
    nfull = num_chunks // _NW
    rem = num_chunks % _NW

    new_w = lax.bitcast_convert_type(
        new_ft.astype(jnp.bfloat16).reshape(N, DW, 2), jnp.int32)
    raw_w = lax.bitcast_convert_type(
        raw_ft.astype(jnp.bfloat16).reshape(N, DW, 2), jnp.int32)

    src = edge_index[0].astype(jnp.int32)
    dst = edge_index[1].astype(jnp.int32)

    mesh = plsc.VectorSubcoreMesh(core_axis_name="c", subcore_axis_name="s")

    @functools.partial(
        pl.kernel,
        mesh=mesh,
        out_type=jax.ShapeDtypeStruct((E,), jnp.float32),
        scratch_types=[
            pltpu.VMEM((_C,), jnp.int32),
            pltpu.VMEM((_C,), jnp.int32),
            pltpu.VMEM((_C,), jnp.int32),
            pltpu.VMEM((_C,), jnp.int32),
            pltpu.VMEM((_C, 128), jnp.int32),
            pltpu.VMEM((_C, 128), jnp.int32),
            pltpu.VMEM((_C, 128), jnp.int32),
            pltpu.VMEM((_C, 128), jnp.int32),
            pltpu.VMEM((_C,), jnp.float32),
            pltpu.SemaphoreType.DMA,
            pltpu.SemaphoreType.DMA,
            pltpu.SemaphoreType.DMA,
            pltpu.SemaphoreType.DMA,
        ],
    )
    def sc_kernel(new_hbm, raw_hbm, src_hbm, dst_hbm, out_hbm,
                  sidx0, didx0, sidx1, didx1,
                  urows0, vrows0, urows1, vrows1, obuf,
                  su0, sv0, su1, sv1):
        wid = lax.axis_index("s") * 2 + lax.axis_index("c")
        n_me = jnp.where(wid < rem, nfull + 1, nfull) if rem else nfull

        bufs = ((sidx0, didx0, urows0, vrows0, su0, sv0),
                (sidx1, didx1, urows1, vrows1, su1, sv1))

        def start_gathers(i):
            base = (wid + i * _NW) * _C

            def go(sidx, didx, ub, vb, su, sv):
                pltpu.sync_copy(src_hbm.at[pl.ds(base, _C)], sidx)
                pltpu.sync_copy(dst_hbm.at[pl.ds(base, _C)], didx)
                pltpu.make_async_copy(new_hbm.at[sidx], ub, su).start()
                pltpu.make_async_copy(raw_hbm.at[didx], vb, sv).start()

            for b in range(2):
                @pl.when(i % 2 == b)
                def _(b=b):
                    go(*bufs[b])

        def body(i, carry):
            @pl.when(i + 1 < n_me)
            def _():
                start_gathers(i + 1)

            base = (wid + i * _NW) * _C
            for b in range(2):
                @pl.when(i % 2 == b)
                def _(b=b):
                    sidx, didx, ub, vb, su, sv = bufs[b]
                    pltpu.make_async_copy(new_hbm.at[sidx], ub, su).wait()
                    pltpu.make_async_copy(raw_hbm.at[didx], vb, sv).wait()
                    _dot_chunk(ub, vb, obuf, DW)
            pltpu.sync_copy(obuf, out_hbm.at[pl.ds(base, _C)])
            return carry

        start_gathers(0)
        lax.fori_loop(0, n_me, body, 0)

    out = sc_kernel(new_w, raw_w, src, dst)
    return out.reshape(E, 1)


# speedup vs baseline: 2.0963x; 2.0963x over previous
import functools

import jax
import jax.numpy as jnp
from jax import lax
from jax.experimental import pallas as pl
from jax.experimental.pallas import tpu as pltpu
from jax.experimental.pallas import tpu_sc as plsc

_C = 128
_NW = 32
_L = 16
_HI = -65536

_DNUMS = lax.GatherDimensionNumbers(
    offset_dims=(), collapsed_slice_dims=(0,), start_index_map=(0,))


def _lane_shuffle(v, perm):
    return lax.gather(v, perm[:, None], _DNUMS, slice_sizes=(1,),
                      mode=lax.GatherScatterMode.PROMISE_IN_BOUNDS)


def _f32(x):
    return lax.bitcast_convert_type(x, jnp.float32)


def _dot_chunk(urows, vrows, obuf, DW):
    lane = lax.iota(jnp.int32, _L)

    def g_body(g, carry):
        row0 = g * _L

        def e_body(e, tot):
            row = row0 + 2 * e
            for k in range(2):
                r = row + k
                uw = urows[r, pl.ds(0, _L)]
                vw = vrows[r, pl.ds(0, _L)]
                acc0 = _f32(uw << 16) * _f32(vw << 16)
                acc1 = _f32(uw & _HI) * _f32(vw & _HI)
                for j in range(1, DW // _L):
                    uw = urows[r, pl.ds(j * _L, _L)]
                    vw = vrows[r, pl.ds(j * _L, _L)]
                    acc0 = acc0 + _f32(uw << 16) * _f32(vw << 16)
                    acc1 = acc1 + _f32(uw & _HI) * _f32(vw & _HI)
                acc = acc0 + acc1
                for s in (1, 2, 4, 8):
                    acc = acc + _lane_shuffle(acc, lane ^ s)
                tot = jnp.where(lane == 2 * e + k, acc, tot)
            return tot

        tot = lax.fori_loop(0, _L // 2, e_body, jnp.zeros((_L,), jnp.float32))
        obuf[pl.ds(row0, _L)] = tot
        return carry

    lax.fori_loop(0, _C // _L, g_body, 0)


def kernel(new_ft, raw_ft, edge_index):
    N, D = new_ft.shape
    E = edge_index.shape[1]
    DW = D // 2
    assert E % _C == 0
    num_chunks = E // _C
    nfull = num_chunks // _NW
    rem = num_chunks % _NW

    def _pack(x):
        xi = lax.bitcast_convert_type(x, jnp.int32)
        t = xi + 0x7FFF + ((xi >> 16) & 1)
        lo = (t[:, :DW] >> 16) & 0xFFFF
        hi = t[:, DW:] & _HI
        return hi | lo

    new_w = _pack(new_ft)
    raw_w = _pack(raw_ft)

    src = edge_index[0].astype(jnp.int32)
    dst = edge_index[1].astype(jnp.int32)

    mesh = plsc.VectorSubcoreMesh(core_axis_name="c", subcore_axis_name="s")

    @functools.partial(
        pl.kernel,
        mesh=mesh,
        out_type=jax.ShapeDtypeStruct((E,), jnp.float32),
        scratch_types=[
            pltpu.VMEM((_C,), jnp.int32),
            pltpu.VMEM((_C,), jnp.int32),
            pltpu.VMEM((_C,), jnp.int32),
            pltpu.VMEM((_C,), jnp.int32),
            pltpu.VMEM((_C, 128), jnp.int32),
            pltpu.VMEM((_C, 128), jnp.int32),
            pltpu.VMEM((_C, 128), jnp.int32),
            pltpu.VMEM((_C, 128), jnp.int32),
            pltpu.VMEM((_C,), jnp.float32),
            pltpu.SemaphoreType.DMA,
            pltpu.SemaphoreType.DMA,
            pltpu.SemaphoreType.DMA,
            pltpu.SemaphoreType.DMA,
        ],
    )
    def sc_kernel(new_hbm, raw_hbm, src_hbm, dst_hbm, out_hbm,
                  sidx0, didx0, sidx1, didx1,
                  urows0, vrows0, urows1, vrows1, obuf,
                  su0, sv0, su1, sv1):
        wid = lax.axis_index("s") * 2 + lax.axis_index("c")
        n_me = jnp.where(wid < rem, nfull + 1, nfull) if rem else nfull

        bufs = ((sidx0, didx0, urows0, vrows0, su0, sv0),
                (sidx1, didx1, urows1, vrows1, su1, sv1))

        def start_gathers(i):
            base = (wid + i * _NW) * _C

            def go(sidx, didx, ub, vb, su, sv):
                pltpu.sync_copy(src_hbm.at[pl.ds(base, _C)], sidx)
                pltpu.sync_copy(dst_hbm.at[pl.ds(base, _C)], didx)
                pltpu.make_async_copy(new_hbm.at[sidx], ub, su).start()
                pltpu.make_async_copy(raw_hbm.at[didx], vb, sv).start()

            for b in range(2):
                @pl.when(i % 2 == b)
                def _(b=b):
                    go(*bufs[b])

        def body(i, carry):
            @pl.when(i + 1 < n_me)
            def _():
                start_gathers(i + 1)

            base = (wid + i * _NW) * _C
            for b in range(2):
                @pl.when(i % 2 == b)
                def _(b=b):
                    sidx, didx, ub, vb, su, sv = bufs[b]
                    pltpu.make_async_copy(new_hbm.at[sidx], ub, su).wait()
                    pltpu.make_async_copy(raw_hbm.at[didx], vb, sv).wait()
                    _dot_chunk(ub, vb, obuf, DW)
            pltpu.sync_copy(obuf, out_hbm.at[pl.ds(base, _C)])
            return carry

        start_gathers(0)
        lax.fori_loop(0, n_me, body, 0)

    out = sc_kernel(new_w, raw_w, src, dst)
    return out.reshape(E, 1)


# speedup vs baseline: 2.3810x; 1.1358x over previous
import functools

import jax
import jax.numpy as jnp
from jax import lax
from jax.experimental import pallas as pl
from jax.experimental.pallas import tpu as pltpu
from jax.experimental.pallas import tpu_sc as plsc

_C = 128
_NW = 32
_L = 16
_HI = -65536

_DNUMS = lax.GatherDimensionNumbers(
    offset_dims=(), collapsed_slice_dims=(0,), start_index_map=(0,))


def _lane_shuffle(v, perm):
    return lax.gather(v, perm[:, None], _DNUMS, slice_sizes=(1,),
                      mode=lax.GatherScatterMode.PROMISE_IN_BOUNDS)


def _f32(x):
    return lax.bitcast_convert_type(x, jnp.float32)


def _dot_chunk(urows, vrows, obuf, DW):
    lane = lax.iota(jnp.int32, _L)

    def g_body(g, carry):
        row0 = g * _L

        def e_body(e, tot):
            row = row0 + 2 * e
            for k in range(2):
                r = row + k
                uw = urows[r, pl.ds(0, _L)]
                vw = vrows[r, pl.ds(0, _L)]
                acc0 = _f32(uw << 16) * _f32(vw << 16)
                acc1 = _f32(uw) * _f32(vw)
                for j in range(1, DW // _L):
                    uw = urows[r, pl.ds(j * _L, _L)]
                    vw = vrows[r, pl.ds(j * _L, _L)]
                    acc0 = acc0 + _f32(uw << 16) * _f32(vw << 16)
                    acc1 = acc1 + _f32(uw) * _f32(vw)
                acc = acc0 + acc1
                for s in (1, 2, 4, 8):
                    acc = acc + _lane_shuffle(acc, lane ^ s)
                tot = jnp.where(lane == 2 * e + k, acc, tot)
            return tot

        tot = lax.fori_loop(0, _L // 2, e_body, jnp.zeros((_L,), jnp.float32))
        obuf[pl.ds(row0, _L)] = tot
        return carry

    lax.fori_loop(0, _C // _L, g_body, 0)


def kernel(new_ft, raw_ft, edge_index):
    N, D = new_ft.shape
    E = edge_index.shape[1]
    DW = D // 2
    assert E % _C == 0
    num_chunks = E // _C
    nfull = num_chunks // _NW
    rem = num_chunks % _NW

    def _pack(x):
        xi = lax.bitcast_convert_type(x, jnp.int32)
        t = xi + 0x7FFF + ((xi >> 16) & 1)
        lo = (t[:, :DW] >> 16) & 0xFFFF
        hi = t[:, DW:] & _HI
        return hi | lo

    new_w = _pack(new_ft)
    raw_w = _pack(raw_ft)

    src = edge_index[0].astype(jnp.int32)
    dst = edge_index[1].astype(jnp.int32)

    mesh = plsc.VectorSubcoreMesh(core_axis_name="c", subcore_axis_name="s")

    @functools.partial(
        pl.kernel,
        mesh=mesh,
        out_type=jax.ShapeDtypeStruct((E,), jnp.float32),
        scratch_types=[
            pltpu.VMEM((_C,), jnp.int32),
            pltpu.VMEM((_C,), jnp.int32),
            pltpu.VMEM((_C,), jnp.int32),
            pltpu.VMEM((_C,), jnp.int32),
            pltpu.VMEM((_C, 128), jnp.int32),
            pltpu.VMEM((_C, 128), jnp.int32),
            pltpu.VMEM((_C, 128), jnp.int32),
            pltpu.VMEM((_C, 128), jnp.int32),
            pltpu.VMEM((_C,), jnp.float32),
            pltpu.SemaphoreType.DMA,
            pltpu.SemaphoreType.DMA,
            pltpu.SemaphoreType.DMA,
            pltpu.SemaphoreType.DMA,
        ],
    )
    def sc_kernel(new_hbm, raw_hbm, src_hbm, dst_hbm, out_hbm,
                  sidx0, didx0, sidx1, didx1,
                  urows0, vrows0, urows1, vrows1, obuf,
                  su0, sv0, su1, sv1):
        wid = lax.axis_index("s") * 2 + lax.axis_index("c")
        n_me = jnp.where(wid < rem, nfull + 1, nfull) if rem else nfull

        bufs = ((sidx0, didx0, urows0, vrows0, su0, sv0),
                (sidx1, didx1, urows1, vrows1, su1, sv1))

        def start_gathers(i):
            base = (wid + i * _NW) * _C

            def go(sidx, didx, ub, vb, su, sv):
                pltpu.sync_copy(src_hbm.at[pl.ds(base, _C)], sidx)
                pltpu.sync_copy(dst_hbm.at[pl.ds(base, _C)], didx)
                pltpu.make_async_copy(new_hbm.at[sidx], ub, su).start()
                pltpu.make_async_copy(raw_hbm.at[didx], vb, sv).start()

            for b in range(2):
                @pl.when(i % 2 == b)
                def _(b=b):
                    go(*bufs[b])

        def body(i, carry):
            @pl.when(i + 1 < n_me)
            def _():
                start_gathers(i + 1)

            base = (wid + i * _NW) * _C
            for b in range(2):
                @pl.when(i % 2 == b)
                def _(b=b):
                    sidx, didx, ub, vb, su, sv = bufs[b]
                    pltpu.make_async_copy(new_hbm.at[sidx], ub, su).wait()
                    pltpu.make_async_copy(raw_hbm.at[didx], vb, sv).wait()
                    _dot_chunk(ub, vb, obuf, DW)
            pltpu.sync_copy(obuf, out_hbm.at[pl.ds(base, _C)])
            return carry

        start_gathers(0)
        lax.fori_loop(0, n_me, body, 0)

    out = sc_kernel(new_w, raw_w, src, dst)
    return out.reshape(E, 1)
